# baseline (speedup 1.0000x reference)
import jax
import jax.numpy as jnp
from jax import lax
from jax.experimental import pallas as pl
from jax.experimental.pallas import tpu as pltpu

N = 2048
DIM = 768
H = 12
KD = 32
D = 64
QKV = H * KD * 2 + H * D
CBS = 16
CST = 8
TOPN = 16
WIN = 512
KC = H * KD
VC = H * D
M = (N - CBS) // CST + 1
MP = 256
QT = 256
NT = N // QT
SCALE = KD ** (-0.5)
SCALE_W = (DIM // H) ** (-0.5)
F32 = jnp.float32
NEG = -1e30


def _gelu(x):
    return 0.5 * x * (1.0 + lax.erf(x * 0.7071067811865476))


def _sigmoid(x):
    return 1.0 / (1.0 + jnp.exp(-x))


def _softmax(s):
    m = jnp.max(s, axis=1, keepdims=True)
    e = jnp.exp(s - m)
    return e / jnp.sum(e, axis=1, keepdims=True)


def _dot_nt(a, b):
    return lax.dot_general(a, b, (((1,), (1,)), ((), ())),
                           preferred_element_type=F32)


def _proj_body(x_ref, w_ref, b_ref, o_ref):
    o_ref[...] = jnp.dot(x_ref[...], w_ref[...],
                         preferred_element_type=F32) + b_ref[...]


def _cmp1_body(seg, kf_ref, pos_ref, w1_ref, b1_ref, o_ref):
    accw = jnp.zeros(o_ref.shape, F32)
    for i in range(CBS):
        li = kf_ref[(i // CST):(i // CST) + MP,
                    (i % CST) * seg:((i % CST) + 1) * seg]
        li = li + pos_ref[:, i * seg:(i + 1) * seg]
        w1i = w1_ref[i * seg:(i + 1) * seg, :]
        accw = accw + jnp.dot(li, w1i, preferred_element_type=F32)
    o_ref[...] = _gelu(accw + b1_ref[...])


def _cmp2_body(h_ref, w2_ref, b2_ref, o_ref):
    o_ref[...] = jnp.dot(h_ref[...], w2_ref[...],
                         preferred_element_type=F32) + b2_ref[...]


def _catt_body(qc_ref, qs_ref, ck_ref, cv_ref, ocmp_ref, obm_ref):
    col = lax.broadcasted_iota(jnp.int32, (QT, MP), 1)
    valid = col < M
    imp = jnp.zeros((QT, MP), F32)
    for h in range(H):
        ckh = ck_ref[:, h * KD:(h + 1) * KD]
        qch = qc_ref[:, h * KD:(h + 1) * KD]
        s = jnp.where(valid, _dot_nt(qch, ckh) * SCALE, NEG)
        p = _softmax(s)
        ocmp_ref[:, h * D:(h + 1) * D] = jnp.dot(
            p, cv_ref[:, h * D:(h + 1) * D], preferred_element_type=F32)
        qsh = qs_ref[:, h * KD:(h + 1) * KD]
        s2 = jnp.where(valid, _dot_nt(qsh, ckh) * SCALE, NEG)
        imp = imp + _softmax(s2)
    impm = jnp.where(valid, imp, NEG)
    vals = impm
    for _ in range(TOPN - 1):
        mx = jnp.max(vals, axis=1, keepdims=True)
        vals = jnp.where(vals >= mx, NEG, vals)
    thr = jnp.max(vals, axis=1, keepdims=True)
    shifted = jnp.concatenate(
        [jnp.full((QT, 1), NEG, F32), impm[:, :MP - 1]], axis=1)
    impm2 = jnp.maximum(impm, shifted)
    obm_ref[...] = (impm2 >= thr).astype(F32)


def _slc_body(bm_ref, q_ref, k_ref, v_ref, o_ref,
              km_scr, acc_scr, m_scr, l_scr):
    qt = pl.program_id(0)
    prow = lax.broadcasted_iota(jnp.int32, (32, QT), 0)
    jcol = lax.broadcasted_iota(jnp.int32, (32, QT), 1)
    e = (jcol // 8 == prow).astype(F32)
    for kt in range(NT):
        mb = bm_ref[:, kt * 32:(kt + 1) * 32]
        km_scr[:, kt * QT:(kt + 1) * QT] = jnp.dot(
            mb, e, preferred_element_type=F32)

    rows = qt * QT + lax.broadcasted_iota(jnp.int32, (QT, QT), 0)
    for h in range(H):
        acc_scr[...] = jnp.zeros_like(acc_scr)
        m_scr[...] = jnp.full_like(m_scr, NEG)
        l_scr[...] = jnp.zeros_like(l_scr)
        q = q_ref[:, h * KD:(h + 1) * KD]
        for kt in range(NT):
            @pl.when(kt <= qt)
            def _step(kt=kt, q=q):
                k = k_ref[kt * QT:(kt + 1) * QT, h * KD:(h + 1) * KD]
                s = _dot_nt(q, k) * SCALE
                cols = kt * QT + lax.broadcasted_iota(jnp.int32, (QT, QT), 1)
                keep = ((cols <= rows)
                        & (km_scr[:, kt * QT:(kt + 1) * QT] > 0.5))
                s = jnp.where(keep, s, NEG)
                mkt = jnp.max(s, axis=1, keepdims=True)
                m_new = jnp.maximum(m_scr[...], mkt)
                pkt = jnp.where(keep, jnp.exp(s - m_new), 0.0)
                alpha = jnp.exp(m_scr[...] - m_new)
                l_scr[...] = l_scr[...] * alpha + jnp.sum(pkt, axis=1,
                                                          keepdims=True)
                acc_scr[...] = acc_scr[...] * alpha + jnp.dot(
                    pkt, v_ref[kt * QT:(kt + 1) * QT, h * D:(h + 1) * D],
                    preferred_element_type=F32)
                m_scr[...] = m_new
        l = l_scr[...]
        o_ref[:, h * D:(h + 1) * D] = jnp.where(
            l > 0.0, acc_scr[...] / jnp.where(l > 0.0, l, 1.0), 0.0)


def _win_body(q_ref, k_ref, v_ref, o_ref, acc_scr, m_scr, l_scr):
    qt = pl.program_id(0)
    rows = qt * QT + lax.broadcasted_iota(jnp.int32, (QT, QT), 0)
    for h in range(H):
        acc_scr[...] = jnp.zeros_like(acc_scr)
        m_scr[...] = jnp.full_like(m_scr, NEG)
        l_scr[...] = jnp.zeros_like(l_scr)
        q = q_ref[:, h * KD:(h + 1) * KD]
        for dd in range(3):
            kt = qt - 2 + dd

            @pl.when(kt >= 0)
            def _step(kt=kt, q=q):
                k = k_ref[pl.ds(kt * QT, QT), h * KD:(h + 1) * KD]
                s = _dot_nt(q, k) * SCALE_W
                cols = kt * QT + lax.broadcasted_iota(jnp.int32, (QT, QT), 1)
                keep = (cols <= rows) & (cols > rows - WIN)
                s = jnp.where(keep, s, NEG)
                mkt = jnp.max(s, axis=1, keepdims=True)
                m_new = jnp.maximum(m_scr[...], mkt)
                pkt = jnp.where(keep, jnp.exp(s - m_new), 0.0)
                alpha = jnp.exp(m_scr[...] - m_new)
                l_scr[...] = l_scr[...] * alpha + jnp.sum(pkt, axis=1,
                                                          keepdims=True)
                acc_scr[...] = acc_scr[...] * alpha + jnp.dot(
                    pkt, v_ref[pl.ds(kt * QT, QT), h * D:(h + 1) * D],
                    preferred_element_type=F32)
                m_scr[...] = m_new
        l = l_scr[...]
        o_ref[:, h * D:(h + 1) * D] = jnp.where(
            l > 0.0, acc_scr[...] / jnp.where(l > 0.0, l, 1.0), 0.0)


def _fin_body(x_ref, gw1_ref, gb1_ref, gw2_ref, gb2_ref,
              cmp_ref, slc_ref, win_ref, pw_ref, pb_ref, o_ref):
    gh = _gelu(jnp.dot(x_ref[...], gw1_ref[...],
                       preferred_element_type=F32) + gb1_ref[...])
    g = _sigmoid(jnp.dot(gh, gw2_ref[...],
                         preferred_element_type=F32) + gb2_ref[...])
    comb = (g[:, 0:1] * cmp_ref[...] + g[:, 1:2] * slc_ref[...]
            + g[:, 2:3] * win_ref[...])
    o_ref[...] = jnp.dot(comb, pw_ref[...],
                         preferred_element_type=F32) + pb_ref[...]


def _full(shape, imap):
    return pl.BlockSpec(shape, imap)


def kernel(x, W_cmp, b_cmp, W_slc, b_slc, W_win, b_win,
           k_pos, k_W1, k_b1, k_W2, k_b2,
           v_pos, v_W1, v_b1, v_W2, v_b2,
           g_W1, g_b1, g_W2, g_b2, p_W, p_b):
    x2 = x[0]
    Wall = jnp.concatenate([W_cmp, W_slc, W_win], axis=1)
    ball = jnp.concatenate([b_cmp, b_slc, b_win])[None, :]

    qkv = pl.pallas_call(
        _proj_body,
        grid=(6,),
        in_specs=[
            _full((N, DIM), lambda ct: (0, 0)),
            _full((DIM, 768), lambda ct: (0, ct)),
            _full((1, 768), lambda ct: (0, ct)),
        ],
        out_specs=_full((N, 768), lambda ct: (0, ct)),
        out_shape=jax.ShapeDtypeStruct((N, 3 * QKV), F32),
    )(x2, Wall, ball)

    kflat = qkv[:, KC:2 * KC]
    vflat = qkv[:, 2 * KC:QKV]
    kfr = jnp.pad(kflat, ((0, 8), (0, 0))).reshape(MP + 1, 8 * KC)
    vfr = jnp.pad(vflat, ((0, 8), (0, 0))).reshape(MP + 1, 8 * VC)
    kposf = k_pos.reshape(1, CBS * KC)
    vposf = v_pos.reshape(1, CBS * VC)

    hk = pl.pallas_call(
        lambda *a: _cmp1_body(KC, *a),
        grid=(1,),
        in_specs=[
            _full((MP + 1, 8 * KC), lambda i: (0, 0)),
            _full((1, CBS * KC), lambda i: (0, 0)),
            _full((CBS * KC, 2 * KC), lambda i: (0, 0)),
            _full((1, 2 * KC), lambda i: (0, 0)),
        ],
        out_specs=_full((MP, 2 * KC), lambda i: (0, 0)),
        out_shape=jax.ShapeDtypeStruct((MP, 2 * KC), F32),
    )(kfr, kposf, k_W1, k_b1[None, :])

    ck = pl.pallas_call(
        _cmp2_body,
        grid=(1,),
        in_specs=[
            _full((MP, 2 * KC), lambda i: (0, 0)),
            _full((2 * KC, KC), lambda i: (0, 0)),
            _full((1, KC), lambda i: (0, 0)),
        ],
        out_specs=_full((MP, KC), lambda i: (0, 0)),
        out_shape=jax.ShapeDtypeStruct((MP, KC), F32),
    )(hk, k_W2, k_b2[None, :])

    hv = pl.pallas_call(
        lambda *a: _cmp1_body(VC, *a),
        grid=(6,),
        in_specs=[
            _full((MP + 1, 8 * VC), lambda ct: (0, 0)),
            _full((1, CBS * VC), lambda ct: (0, 0)),
            _full((CBS * VC, 256), lambda ct: (0, ct)),
            _full((1, 256), lambda ct: (0, ct)),
        ],
        out_specs=_full((MP, 256), lambda ct: (0, ct)),
        out_shape=jax.ShapeDtypeStruct((MP, 2 * VC), F32),
    )(vfr, vposf, v_W1, v_b1[None, :])

    cv = pl.pallas_call(
        _cmp2_body,
        grid=(1,),
        in_specs=[
            _full((MP, 2 * VC), lambda i: (0, 0)),
            _full((2 * VC, VC), lambda i: (0, 0)),
            _full((1, VC), lambda i: (0, 0)),
        ],
        out_specs=_full((MP, VC), lambda i: (0, 0)),
        out_shape=jax.ShapeDtypeStruct((MP, VC), F32),
    )(hv, v_W2, v_b2[None, :])

    out_cmp, bmask = pl.pallas_call(
        _catt_body,
        grid=(NT,),
        in_specs=[
            _full((QT, KC), lambda qt: (qt, 0)),
            _full((QT, KC), lambda qt: (qt, 4)),
            _full((MP, KC), lambda qt: (0, 0)),
            _full((MP, VC), lambda qt: (0, 0)),
        ],
        out_specs=[
            _full((QT, VC), lambda qt: (qt, 0)),
            _full((QT, MP), lambda qt: (qt, 0)),
        ],
        out_shape=[
            jax.ShapeDtypeStruct((N, VC), F32),
            jax.ShapeDtypeStruct((N, MP), F32),
        ],
    )(qkv, qkv, ck, cv)

    out_slc = pl.pallas_call(
        _slc_body,
        grid=(NT,),
        in_specs=[
            _full((QT, MP), lambda qt: (qt, 0)),
            _full((QT, KC), lambda qt: (qt, 4)),
            _full((N, KC), lambda qt: (0, 5)),
            _full((N, VC), lambda qt: (0, 3)),
        ],
        out_specs=_full((QT, VC), lambda qt: (qt, 0)),
        out_shape=jax.ShapeDtypeStruct((N, VC), F32),
        scratch_shapes=[
            pltpu.VMEM((QT, N), F32),
            pltpu.VMEM((QT, D), F32),
            pltpu.VMEM((QT, 1), F32),
            pltpu.VMEM((QT, 1), F32),
        ],
    )(bmask, qkv, qkv, qkv)

    out_win = pl.pallas_call(
        _win_body,
        grid=(NT,),
        in_specs=[
            _full((QT, KC), lambda qt: (qt, 8)),
            _full((N, KC), lambda qt: (0, 9)),
            _full((N, VC), lambda qt: (0, 5)),
        ],
        out_specs=_full((QT, VC), lambda qt: (qt, 0)),
        out_shape=jax.ShapeDtypeStruct((N, VC), F32),
        scratch_shapes=[
            pltpu.VMEM((QT, D), F32),
            pltpu.VMEM((QT, 1), F32),
            pltpu.VMEM((QT, 1), F32),
        ],
    )(qkv, qkv, qkv)

    gW2p = jnp.pad(g_W2, ((0, 0), (0, 125)))
    gb2p = jnp.pad(g_b2, (0, 125))[None, :]
    out = pl.pallas_call(
        _fin_body,
        grid=(NT,),
        in_specs=[
            _full((QT, DIM), lambda qt: (qt, 0)),
            _full((DIM, DIM // 2), lambda qt: (0, 0)),
            _full((1, DIM // 2), lambda qt: (0, 0)),
            _full((DIM // 2, 128), lambda qt: (0, 0)),
            _full((1, 128), lambda qt: (0, 0)),
            _full((QT, VC), lambda qt: (qt, 0)),
            _full((QT, VC), lambda qt: (qt, 0)),
            _full((QT, VC), lambda qt: (qt, 0)),
            _full((VC, DIM), lambda qt: (0, 0)),
            _full((1, DIM), lambda qt: (0, 0)),
        ],
        out_specs=_full((QT, DIM), lambda qt: (qt, 0)),
        out_shape=jax.ShapeDtypeStruct((N, DIM), F32),
    )(x2, g_W1, g_b1[None, :], gW2p, gb2p,
      out_cmp, out_slc, out_win, p_W, p_b[None, :])

    return out[None, :, :]


# speedup vs baseline: 1.6423x; 1.6423x over previous
import jax
import jax.numpy as jnp
from jax import lax
from jax.experimental import pallas as pl
from jax.experimental.pallas import tpu as pltpu

N = 2048
DIM = 768
H = 12
KD = 32
D = 64
QKV = H * KD * 2 + H * D
CBS = 16
CST = 8
TOPN = 16
WIN = 512
KC = H * KD
VC = H * D
M = (N - CBS) // CST + 1
MP = 256
QT = 256
NT = N // QT
SCALE = KD ** (-0.5)
SCALE_W = (DIM // H) ** (-0.5)
F32 = jnp.float32
NEG = -1e30


def _gelu(x):
    return 0.5 * x * (1.0 + lax.erf(x * 0.7071067811865476))


def _sigmoid(x):
    return 1.0 / (1.0 + jnp.exp(-x))


def _softmax(s):
    m = jnp.max(s, axis=1, keepdims=True)
    e = jnp.exp(s - m)
    return e / jnp.sum(e, axis=1, keepdims=True)


BF16 = jnp.bfloat16


def _dot_nt(a, b):
    return lax.dot_general(a.astype(BF16), b.astype(BF16),
                           (((1,), (1,)), ((), ())),
                           preferred_element_type=F32)


def _dotb(a, b):
    return jnp.dot(a.astype(BF16), b.astype(BF16),
                   preferred_element_type=F32)


def _proj_body(x_ref, w_ref, b_ref, o_ref):
    o_ref[...] = _dotb(x_ref[...], w_ref[...]) + b_ref[...]


def _cmp1_body(seg, kf_ref, pos_ref, w1_ref, b1_ref, o_ref):
    accw = jnp.zeros(o_ref.shape, F32)
    for i in range(CBS):
        li = kf_ref[(i // CST):(i // CST) + MP,
                    (i % CST) * seg:((i % CST) + 1) * seg]
        li = li + pos_ref[:, i * seg:(i + 1) * seg]
        w1i = w1_ref[i * seg:(i + 1) * seg, :]
        accw = accw + _dotb(li, w1i)
    o_ref[...] = _gelu(accw + b1_ref[...])


def _cmp2_body(h_ref, w2_ref, b2_ref, o_ref):
    o_ref[...] = _dotb(h_ref[...], w2_ref[...]) + b2_ref[...]


def _catt_body(qc_ref, qs_ref, ck_ref, cv_ref, ocmp_ref, obm_ref):
    col = lax.broadcasted_iota(jnp.int32, (QT, MP), 1)
    valid = col < M
    imp = jnp.zeros((QT, MP), F32)
    for h in range(H):
        ckh = ck_ref[:, h * KD:(h + 1) * KD]
        qch = qc_ref[:, h * KD:(h + 1) * KD]
        s = jnp.where(valid, _dot_nt(qch, ckh) * SCALE, NEG)
        p = _softmax(s)
        ocmp_ref[:, h * D:(h + 1) * D] = _dotb(
            p, cv_ref[:, h * D:(h + 1) * D])
        qsh = qs_ref[:, h * KD:(h + 1) * KD]
        s2 = jnp.where(valid, _dot_nt(qsh, ckh) * SCALE, NEG)
        imp = imp + _softmax(s2)
    impm = jnp.where(valid, imp, NEG)
    vals = impm
    for _ in range(TOPN - 1):
        mx = jnp.max(vals, axis=1, keepdims=True)
        vals = jnp.where(vals >= mx, NEG, vals)
    thr = jnp.max(vals, axis=1, keepdims=True)
    shifted = jnp.concatenate(
        [jnp.full((QT, 1), NEG, F32), impm[:, :MP - 1]], axis=1)
    impm2 = jnp.maximum(impm, shifted)
    obm_ref[...] = (impm2 >= thr).astype(F32)


def _slc_body(bm_ref, q_ref, k_ref, v_ref, o_ref,
              madd_scr, sc_scr, pb_scr):
    qt = pl.program_id(0)
    rows = qt * QT + lax.broadcasted_iota(jnp.int32, (QT, QT), 0)
    prow = lax.broadcasted_iota(jnp.int32, (32, QT), 0)
    jcol = lax.broadcasted_iota(jnp.int32, (32, QT), 1)
    e = (jcol // 8 == prow).astype(F32)
    for kt in range(NT):
        @pl.when(kt <= qt)
        def _mk(kt=kt):
            km = _dotb(bm_ref[:, kt * 32:(kt + 1) * 32], e)
            cols = kt * QT + lax.broadcasted_iota(jnp.int32, (QT, QT), 1)
            keep = (cols <= rows) & (km > 0.5)
            madd_scr[:, kt * QT:(kt + 1) * QT] = jnp.where(keep, 0.0, NEG)

        @pl.when(kt > qt)
        def _mi(kt=kt):
            sc_scr[:, kt * QT:(kt + 1) * QT] = jnp.full((QT, QT), NEG, F32)

    for h in range(H):
        q = q_ref[:, h * KD:(h + 1) * KD]
        for kt in range(NT):
            @pl.when(kt <= qt)
            def _qk(kt=kt, q=q):
                k = k_ref[kt * QT:(kt + 1) * QT, h * KD:(h + 1) * KD]
                sc_scr[:, kt * QT:(kt + 1) * QT] = (
                    _dot_nt(q, k) * SCALE
                    + madd_scr[:, kt * QT:(kt + 1) * QT])
        sc = sc_scr[...]
        m = jnp.max(sc, axis=1, keepdims=True)
        el = jnp.exp(sc - m)
        l = jnp.sum(el, axis=1, keepdims=True)
        inv = jnp.where(m > -1e29, 1.0 / l, 0.0)
        pb_scr[...] = (el * inv).astype(BF16)
        o_ref[:, h * D:(h + 1) * D] = jnp.dot(
            pb_scr[...], v_ref[:, h * D:(h + 1) * D].astype(BF16),
            preferred_element_type=F32)


def _win_body(q_ref, k_ref, v_ref, o_ref, madd_scr, sc_scr, pb_scr):
    qt = pl.program_id(0)
    rows = qt * QT + lax.broadcasted_iota(jnp.int32, (QT, QT), 0)
    for dd in range(3):
        kt = qt - 2 + dd
        cols = kt * QT + lax.broadcasted_iota(jnp.int32, (QT, QT), 1)
        keep = (cols <= rows) & (cols > rows - WIN) & (kt >= 0)
        madd_scr[:, dd * QT:(dd + 1) * QT] = jnp.where(keep, 0.0, NEG)
    for h in range(H):
        q = q_ref[:, h * KD:(h + 1) * KD]
        for dd in range(3):
            kt = jnp.maximum(qt - 2 + dd, 0)
            k = k_ref[pl.ds(kt * QT, QT), h * KD:(h + 1) * KD]
            sc_scr[:, dd * QT:(dd + 1) * QT] = (
                _dot_nt(q, k) * SCALE_W
                + madd_scr[:, dd * QT:(dd + 1) * QT])
        sc = sc_scr[...]
        m = jnp.max(sc, axis=1, keepdims=True)
        el = jnp.exp(sc - m)
        l = jnp.sum(el, axis=1, keepdims=True)
        pb_scr[...] = (el / l).astype(BF16)
        acc = jnp.zeros((QT, D), F32)
        for dd in range(3):
            kt = jnp.maximum(qt - 2 + dd, 0)
            v = v_ref[pl.ds(kt * QT, QT), h * D:(h + 1) * D]
            acc = acc + jnp.dot(pb_scr[:, dd * QT:(dd + 1) * QT],
                                v.astype(BF16), preferred_element_type=F32)
        o_ref[:, h * D:(h + 1) * D] = acc


def _fin_body(x_ref, gw1_ref, gb1_ref, gw2_ref, gb2_ref,
              cmp_ref, slc_ref, win_ref, pw_ref, pb_ref, o_ref):
    gh = _gelu(_dotb(x_ref[...], gw1_ref[...]) + gb1_ref[...])
    g = _sigmoid(_dotb(gh, gw2_ref[...]) + gb2_ref[...])
    comb = (g[:, 0:1] * cmp_ref[...] + g[:, 1:2] * slc_ref[...]
            + g[:, 2:3] * win_ref[...])
    o_ref[...] = _dotb(comb, pw_ref[...]) + pb_ref[...]


def _full(shape, imap):
    return pl.BlockSpec(shape, imap)


def kernel(x, W_cmp, b_cmp, W_slc, b_slc, W_win, b_win,
           k_pos, k_W1, k_b1, k_W2, k_b2,
           v_pos, v_W1, v_b1, v_W2, v_b2,
           g_W1, g_b1, g_W2, g_b2, p_W, p_b):
    x2 = x[0]
    Wall = jnp.concatenate([W_cmp, W_slc, W_win], axis=1)
    ball = jnp.concatenate([b_cmp, b_slc, b_win])[None, :]

    qkv = pl.pallas_call(
        _proj_body,
        grid=(6,),
        in_specs=[
            _full((N, DIM), lambda ct: (0, 0)),
            _full((DIM, 768), lambda ct: (0, ct)),
            _full((1, 768), lambda ct: (0, ct)),
        ],
        out_specs=_full((N, 768), lambda ct: (0, ct)),
        out_shape=jax.ShapeDtypeStruct((N, 3 * QKV), F32),
    )(x2, Wall, ball)

    kflat = qkv[:, KC:2 * KC]
    vflat = qkv[:, 2 * KC:QKV]
    kfr = jnp.pad(kflat, ((0, 8), (0, 0))).reshape(MP + 1, 8 * KC)
    vfr = jnp.pad(vflat, ((0, 8), (0, 0))).reshape(MP + 1, 8 * VC)
    kposf = k_pos.reshape(1, CBS * KC)
    vposf = v_pos.reshape(1, CBS * VC)

    hk = pl.pallas_call(
        lambda *a: _cmp1_body(KC, *a),
        grid=(1,),
        in_specs=[
            _full((MP + 1, 8 * KC), lambda i: (0, 0)),
            _full((1, CBS * KC), lambda i: (0, 0)),
            _full((CBS * KC, 2 * KC), lambda i: (0, 0)),
            _full((1, 2 * KC), lambda i: (0, 0)),
        ],
        out_specs=_full((MP, 2 * KC), lambda i: (0, 0)),
        out_shape=jax.ShapeDtypeStruct((MP, 2 * KC), F32),
    )(kfr, kposf, k_W1, k_b1[None, :])

    ck = pl.pallas_call(
        _cmp2_body,
        grid=(1,),
        in_specs=[
            _full((MP, 2 * KC), lambda i: (0, 0)),
            _full((2 * KC, KC), lambda i: (0, 0)),
            _full((1, KC), lambda i: (0, 0)),
        ],
        out_specs=_full((MP, KC), lambda i: (0, 0)),
        out_shape=jax.ShapeDtypeStruct((MP, KC), F32),
    )(hk, k_W2, k_b2[None, :])

    hv = pl.pallas_call(
        lambda *a: _cmp1_body(VC, *a),
        grid=(6,),
        in_specs=[
            _full((MP + 1, 8 * VC), lambda ct: (0, 0)),
            _full((1, CBS * VC), lambda ct: (0, 0)),
            _full((CBS * VC, 256), lambda ct: (0, ct)),
            _full((1, 256), lambda ct: (0, ct)),
        ],
        out_specs=_full((MP, 256), lambda ct: (0, ct)),
        out_shape=jax.ShapeDtypeStruct((MP, 2 * VC), F32),
    )(vfr, vposf, v_W1, v_b1[None, :])

    cv = pl.pallas_call(
        _cmp2_body,
        grid=(1,),
        in_specs=[
            _full((MP, 2 * VC), lambda i: (0, 0)),
            _full((2 * VC, VC), lambda i: (0, 0)),
            _full((1, VC), lambda i: (0, 0)),
        ],
        out_specs=_full((MP, VC), lambda i: (0, 0)),
        out_shape=jax.ShapeDtypeStruct((MP, VC), F32),
    )(hv, v_W2, v_b2[None, :])

    out_cmp, bmask = pl.pallas_call(
        _catt_body,
        grid=(NT,),
        in_specs=[
            _full((QT, KC), lambda qt: (qt, 0)),
            _full((QT, KC), lambda qt: (qt, 4)),
            _full((MP, KC), lambda qt: (0, 0)),
            _full((MP, VC), lambda qt: (0, 0)),
        ],
        out_specs=[
            _full((QT, VC), lambda qt: (qt, 0)),
            _full((QT, MP), lambda qt: (qt, 0)),
        ],
        out_shape=[
            jax.ShapeDtypeStruct((N, VC), F32),
            jax.ShapeDtypeStruct((N, MP), F32),
        ],
    )(qkv, qkv, ck, cv)

    out_slc = pl.pallas_call(
        _slc_body,
        grid=(NT,),
        in_specs=[
            _full((QT, MP), lambda qt: (qt, 0)),
            _full((QT, KC), lambda qt: (qt, 4)),
            _full((N, KC), lambda qt: (0, 5)),
            _full((N, VC), lambda qt: (0, 3)),
        ],
        out_specs=_full((QT, VC), lambda qt: (qt, 0)),
        out_shape=jax.ShapeDtypeStruct((N, VC), F32),
        scratch_shapes=[
            pltpu.VMEM((QT, N), F32),
            pltpu.VMEM((QT, N), F32),
            pltpu.VMEM((QT, N), BF16),
        ],
    )(bmask, qkv, qkv, qkv)

    out_win = pl.pallas_call(
        _win_body,
        grid=(NT,),
        in_specs=[
            _full((QT, KC), lambda qt: (qt, 8)),
            _full((N, KC), lambda qt: (0, 9)),
            _full((N, VC), lambda qt: (0, 5)),
        ],
        out_specs=_full((QT, VC), lambda qt: (qt, 0)),
        out_shape=jax.ShapeDtypeStruct((N, VC), F32),
        scratch_shapes=[
            pltpu.VMEM((QT, 3 * QT), F32),
            pltpu.VMEM((QT, 3 * QT), F32),
            pltpu.VMEM((QT, 3 * QT), BF16),
        ],
    )(qkv, qkv, qkv)

    gW2p = jnp.pad(g_W2, ((0, 0), (0, 125)))
    gb2p = jnp.pad(g_b2, (0, 125))[None, :]
    out = pl.pallas_call(
        _fin_body,
        grid=(NT,),
        in_specs=[
            _full((QT, DIM), lambda qt: (qt, 0)),
            _full((DIM, DIM // 2), lambda qt: (0, 0)),
            _full((1, DIM // 2), lambda qt: (0, 0)),
            _full((DIM // 2, 128), lambda qt: (0, 0)),
            _full((1, 128), lambda qt: (0, 0)),
            _full((QT, VC), lambda qt: (qt, 0)),
            _full((QT, VC), lambda qt: (qt, 0)),
            _full((QT, VC), lambda qt: (qt, 0)),
            _full((VC, DIM), lambda qt: (0, 0)),
            _full((1, DIM), lambda qt: (0, 0)),
        ],
        out_specs=_full((QT, DIM), lambda qt: (qt, 0)),
        out_shape=jax.ShapeDtypeStruct((N, DIM), F32),
    )(x2, g_W1, g_b1[None, :], gW2p, gb2p,
      out_cmp, out_slc, out_win, p_W, p_b[None, :])

    return out[None, :, :]
